# baseline (speedup 1.0000x reference)
import functools
import jax
import jax.numpy as jnp
from jax import lax
from jax.experimental import pallas as pl
from jax.experimental.pallas import tpu as pltpu
from jax.experimental.pallas import tpu_sc as plsc

_B = 16384
_N = 1003
_NOUT = 1000
_NPAD = 1008
_NW = 32
_SC_ROWS = 7168
_TC_ROWS = _B - _SC_ROWS
_TC_BLK = 512
_RPW = _SC_ROWS // _NW
_CHUNK = 16
_NCH = _RPW // _CHUNK

_LN2 = 0.6931471805599453
_SQRT2 = 1.4142135623730951


def _vlog(v):
    bits = lax.bitcast_convert_type(v, jnp.int32)
    e = (bits >> 23) - 127
    m = lax.bitcast_convert_type((bits & 0x007FFFFF) | 0x3F800000, jnp.float32)
    big = m > _SQRT2
    m = jnp.where(big, m * 0.5, m)
    e = e + jnp.where(big, 1, 0)
    t = (m - 1.0) / (m + 1.0)
    t2 = t * t
    p = t * (2.0 + t2 * (2.0 / 3.0 + t2 * (2.0 / 5.0 + t2 * (2.0 / 7.0 + t2 * (2.0 / 9.0)))))
    return e.astype(jnp.float32) * _LN2 + p


_A_OFFS = [16 * k for k in range(62)] + [987]


def _sc_kernel_body(x_hbm, t_hbm, b_hbm, out_hbm,
                    xb0, xb1, ob0, ob1, tbuf, bbuf,
                    isem0, isem1, osem0, osem1):
    wid = lax.axis_index("s") * 2 + lax.axis_index("c")
    pltpu.sync_copy(t_hbm, tbuf)
    pltpu.sync_copy(b_hbm, bbuf)
    lane = lax.iota(jnp.int32, 16)
    xbufs = (xb0, xb1)
    obufs = (ob0, ob1)
    isems = (isem0, isem1)
    osems = (osem0, osem1)

    def in_src(g):
        row0 = wid * _RPW + g * _CHUNK
        return x_hbm.at[pl.ds(row0, _CHUNK)]

    def out_dst(g):
        row0 = wid * _RPW + g * _CHUNK
        return out_hbm.at[pl.ds(row0, _CHUNK)]

    def compute_chunk(xbuf, cbuf):
        @plsc.parallel_loop(0, _CHUNK // 2, 1, unroll=1)
        def pair_body(q):
            rA = q
            rB = q + 8

            def lds(k):
                off = _A_OFFS[k]
                return (tbuf[pl.ds(off, 16)], bbuf[pl.ds(off, 16)],
                        xbuf[rA, pl.ds(off, 16)],
                        xbuf[rB, pl.ds(off, 16)])

            accA = [jnp.zeros((16,), jnp.float32) for _ in range(3)]
            accB = [jnp.zeros((16,), jnp.float32) for _ in range(3)]
            elA = jnp.zeros((16,), jnp.float32)
            elB = jnp.zeros((16,), jnp.float32)

            cur = lds(0)
            for k in range(63):
                nxt = lds(k + 1) if k < 62 else cur
                tv, bv, xa, xc = cur
                ea = jnp.exp(xa * tv + bv)
                eb = jnp.exp(xc * tv + bv)
                if k < 24:
                    accA[0] = accA[0] + ea
                    accB[0] = accB[0] + eb
                elif k == 24:
                    accA[0] = accA[0] + jnp.where(lane < 8, ea, 0.0)
                    accA[1] = accA[1] + jnp.where(lane >= 8, ea, 0.0)
                    accB[0] = accB[0] + jnp.where(lane < 8, eb, 0.0)
                    accB[1] = accB[1] + jnp.where(lane >= 8, eb, 0.0)
                    elA = elA + jnp.where(lane == 7, ea, 0.0)
                    elB = elB + jnp.where(lane == 7, eb, 0.0)
                elif k < 54:
                    accA[1] = accA[1] + ea
                    accB[1] = accB[1] + eb
                elif k == 54:
                    accA[1] = accA[1] + jnp.where(lane < 2, ea, 0.0)
                    accA[2] = accA[2] + jnp.where(lane >= 2, ea, 0.0)
                    accB[1] = accB[1] + jnp.where(lane < 2, eb, 0.0)
                    accB[2] = accB[2] + jnp.where(lane >= 2, eb, 0.0)
                    elA = elA + jnp.where(lane == 1, ea, 0.0)
                    elB = elB + jnp.where(lane == 1, eb, 0.0)
                elif k < 62:
                    accA[2] = accA[2] + ea
                    accB[2] = accB[2] + eb
                else:
                    accA[2] = accA[2] + jnp.where(lane >= 5, ea, 0.0)
                    accB[2] = accB[2] + jnp.where(lane >= 5, eb, 0.0)
                    elA = elA + jnp.where(lane == 15, ea, 0.0)
                    elB = elB + jnp.where(lane == 15, eb, 0.0)
                cur = nxt

            sA1, sA2, sA3 = (jnp.sum(a) for a in accA)
            sB1, sB2, sB3 = (jnp.sum(a) for a in accB)
            svA = jnp.where(lane == 7, sA1, jnp.where(lane == 1, sA2, sA3))
            svB = jnp.where(lane == 7, sB1, jnp.where(lane == 1, sB2, sB3))
            rnA = 3.0 - jnp.sum(elA / svA)
            rnB = 3.0 - jnp.sum(elB / svB)
            packedA = jnp.where(lane == 0, sA1 * rnA,
                                jnp.where(lane == 1, sA2 * rnA, sA3 * rnA))
            packedB = jnp.where(lane == 0, sB1 * rnB,
                                jnp.where(lane == 1, sB2 * rnB, sB3 * rnB))
            cbuf[rA, :] = _vlog(packedA)
            cbuf[rB, :] = _vlog(packedB)

    pltpu.async_copy(in_src(0), xb0, isem0)
    pltpu.async_copy(in_src(1), xb1, isem1)

    def outer(gg, carry):
        for par in range(2):
            g = gg * 2 + par
            xbuf, obuf = xbufs[par], obufs[par]
            isem, osem = isems[par], osems[par]
            pltpu.make_async_copy(in_src(g), xbuf, isem).wait()

            @pl.when(gg >= 1)
            def _():
                pltpu.make_async_copy(obuf, out_dst(g), osem).wait()

            compute_chunk(xbuf, obuf)
            pltpu.async_copy(obuf, out_dst(g), osem)

            @pl.when(gg < (_NCH // 2 - 1))
            def _():
                pltpu.async_copy(in_src(g + 2), xbuf, isem)
        return carry

    lax.fori_loop(0, _NCH // 2, outer, 0)
    pltpu.make_async_copy(ob0, out_dst(_NCH - 2), osem0).wait()
    pltpu.make_async_copy(ob1, out_dst(_NCH - 1), osem1).wait()


def _run_sc(x, t, b):
    mesh = plsc.VectorSubcoreMesh(core_axis_name="c", subcore_axis_name="s")
    f = functools.partial(
        pl.kernel,
        mesh=mesh,
        compiler_params=pltpu.CompilerParams(needs_layout_passes=False),
        out_type=jax.ShapeDtypeStruct((_SC_ROWS, 16), jnp.float32),
        scratch_types=[
            pltpu.VMEM((_CHUNK, _N), jnp.float32),
            pltpu.VMEM((_CHUNK, _N), jnp.float32),
            pltpu.VMEM((_CHUNK, 16), jnp.float32),
            pltpu.VMEM((_CHUNK, 16), jnp.float32),
            pltpu.VMEM((_NPAD,), jnp.float32),
            pltpu.VMEM((_NPAD,), jnp.float32),
            pltpu.SemaphoreType.DMA,
            pltpu.SemaphoreType.DMA,
            pltpu.SemaphoreType.DMA,
            pltpu.SemaphoreType.DMA,
        ],
    )(_sc_kernel_body)
    return f(x, t, b)


def _tc_body(x_ref, t_ref, b_ref, o_ref):
    z = x_ref[...] * t_ref[...] + b_ref[...]
    z1 = z[:, 0:392]
    z2 = z[:, 392:866]
    z3 = z[:, 866:_N]
    m1 = jnp.max(z1, axis=1, keepdims=True)
    m2 = jnp.max(z2, axis=1, keepdims=True)
    m3 = jnp.max(z3, axis=1, keepdims=True)
    e1 = jnp.exp(z1 - m1)
    e2 = jnp.exp(z2 - m2)
    e3 = jnp.exp(z3 - m3)
    s1 = jnp.sum(e1, axis=1, keepdims=True)
    s2 = jnp.sum(e2, axis=1, keepdims=True)
    s3 = jnp.sum(e3, axis=1, keepdims=True)
    renorm = 3.0 - e1[:, -1:] / s1 - e2[:, -1:] / s2 - e3[:, -1:] / s3
    lr = jnp.log(renorm)
    c1 = m1 + jnp.log(s1) + lr
    c2 = m2 + jnp.log(s2) + lr
    c3 = m3 + jnp.log(s3) + lr
    o_ref[:, 0:391] = z1[:, :-1] - c1
    o_ref[:, 391:864] = z2[:, :-1] - c2
    o_ref[:, 864:1000] = z3[:, :-1] - c3


def _run_tc(x, t2, b2):
    nblk = _TC_ROWS // _TC_BLK
    off = _SC_ROWS // _TC_BLK
    return pl.pallas_call(
        _tc_body,
        grid=(nblk,),
        in_specs=[
            pl.BlockSpec((_TC_BLK, _N), lambda i: (i + off, 0)),
            pl.BlockSpec((1, _N), lambda i: (0, 0)),
            pl.BlockSpec((1, _N), lambda i: (0, 0)),
        ],
        out_specs=pl.BlockSpec((_TC_BLK, _NOUT), lambda i: (i + off, 0)),
        out_shape=jax.ShapeDtypeStruct((_B, _NOUT), jnp.float32),
    )(x, t2, b2)


def _tc2_body(full_ref, x_ref, t_ref, b_ref, c_ref, o_ref):
    z = x_ref[...] * t_ref[...] + b_ref[...]
    c1 = c_ref[:, 0:1]
    c2 = c_ref[:, 1:2]
    c3 = c_ref[:, 2:3]
    o_ref[:, 0:391] = z[:, 0:391] - c1
    o_ref[:, 391:864] = z[:, 392:865] - c2
    o_ref[:, 864:1000] = z[:, 866:1002] - c3


def _run_tc2(out_full, x, t2, b2, corr):
    nblk = _SC_ROWS // _TC_BLK
    return pl.pallas_call(
        _tc2_body,
        grid=(nblk,),
        in_specs=[
            pl.BlockSpec(memory_space=pltpu.MemorySpace.HBM),
            pl.BlockSpec((_TC_BLK, _N), lambda i: (i, 0)),
            pl.BlockSpec((1, _N), lambda i: (0, 0)),
            pl.BlockSpec((1, _N), lambda i: (0, 0)),
            pl.BlockSpec((_TC_BLK, 16), lambda i: (i, 0)),
        ],
        out_specs=pl.BlockSpec((_TC_BLK, _NOUT), lambda i: (i, 0)),
        out_shape=jax.ShapeDtypeStruct((_B, _NOUT), jnp.float32),
        input_output_aliases={0: 0},
    )(out_full, x, t2, b2, corr)


@jax.jit
def _run_hybrid(x, t, b, t2, b2):
    corr = _run_sc(x, t, b)
    out_full = _run_tc(x, t2, b2)
    return _run_tc2(out_full, x, t2, b2, corr)


def kernel(x, manyshotTemp, mediumshotTemp, fewshotTemp, manyshotBias,
           mediumshotBias, fewshotBias, many_mask, med_mask, few_mask):
    t2 = jnp.concatenate([manyshotTemp, mediumshotTemp, fewshotTemp], axis=1)
    b2 = jnp.concatenate([manyshotBias, mediumshotBias, fewshotBias], axis=1)
    pad = jnp.zeros((1, _NPAD - _N), jnp.float32)
    t = jnp.concatenate([t2, pad], axis=1)[0]
    b = jnp.concatenate([b2, pad], axis=1)[0]
    return _run_hybrid(x, t, b, t2, b2)


# speedup vs baseline: 1.0025x; 1.0025x over previous
import functools
import jax
import jax.numpy as jnp
from jax import lax
from jax.experimental import pallas as pl
from jax.experimental.pallas import tpu as pltpu
from jax.experimental.pallas import tpu_sc as plsc

_B = 16384
_N = 1003
_NOUT = 1000
_NPAD = 1008
_NW = 32
_SC_ROWS = 8192
_TC_ROWS = _B - _SC_ROWS
_TC_BLK = 512
_RPW = _SC_ROWS // _NW
_CHUNK = 16
_NCH = _RPW // _CHUNK

_LN2 = 0.6931471805599453
_SQRT2 = 1.4142135623730951


def _vlog(v):
    bits = lax.bitcast_convert_type(v, jnp.int32)
    e = (bits >> 23) - 127
    m = lax.bitcast_convert_type((bits & 0x007FFFFF) | 0x3F800000, jnp.float32)
    big = m > _SQRT2
    m = jnp.where(big, m * 0.5, m)
    e = e + jnp.where(big, 1, 0)
    t = (m - 1.0) / (m + 1.0)
    t2 = t * t
    p = t * (2.0 + t2 * (2.0 / 3.0 + t2 * (2.0 / 5.0 + t2 * (2.0 / 7.0 + t2 * (2.0 / 9.0)))))
    return e.astype(jnp.float32) * _LN2 + p


_A_OFFS = [16 * k for k in range(62)] + [987]


def _sc_kernel_body(x_hbm, t_hbm, b_hbm, out_hbm,
                    xb0, xb1, ob0, ob1, tbuf, bbuf,
                    isem0, isem1, osem0, osem1):
    wid = lax.axis_index("s") * 2 + lax.axis_index("c")
    pltpu.sync_copy(t_hbm, tbuf)
    pltpu.sync_copy(b_hbm, bbuf)
    lane = lax.iota(jnp.int32, 16)
    xbufs = (xb0, xb1)
    obufs = (ob0, ob1)
    isems = (isem0, isem1)
    osems = (osem0, osem1)

    def in_src(g):
        row0 = wid * _RPW + g * _CHUNK
        return x_hbm.at[pl.ds(row0, _CHUNK)]

    def out_dst(g):
        row0 = wid * _RPW + g * _CHUNK
        return out_hbm.at[pl.ds(row0, _CHUNK)]

    def compute_chunk(xbuf, cbuf):
        @plsc.parallel_loop(0, _CHUNK // 2, 1, unroll=1)
        def pair_body(q):
            rA = q
            rB = q + 8

            def lds(k):
                off = _A_OFFS[k]
                return (tbuf[pl.ds(off, 16)], bbuf[pl.ds(off, 16)],
                        xbuf[rA, pl.ds(off, 16)],
                        xbuf[rB, pl.ds(off, 16)])

            accA = [jnp.zeros((16,), jnp.float32) for _ in range(3)]
            accB = [jnp.zeros((16,), jnp.float32) for _ in range(3)]
            elA = jnp.zeros((16,), jnp.float32)
            elB = jnp.zeros((16,), jnp.float32)

            cur = lds(0)
            for k in range(63):
                nxt = lds(k + 1) if k < 62 else cur
                tv, bv, xa, xc = cur
                ea = jnp.exp(xa * tv + bv)
                eb = jnp.exp(xc * tv + bv)
                if k < 24:
                    accA[0] = accA[0] + ea
                    accB[0] = accB[0] + eb
                elif k == 24:
                    accA[0] = accA[0] + jnp.where(lane < 8, ea, 0.0)
                    accA[1] = accA[1] + jnp.where(lane >= 8, ea, 0.0)
                    accB[0] = accB[0] + jnp.where(lane < 8, eb, 0.0)
                    accB[1] = accB[1] + jnp.where(lane >= 8, eb, 0.0)
                    elA = elA + jnp.where(lane == 7, ea, 0.0)
                    elB = elB + jnp.where(lane == 7, eb, 0.0)
                elif k < 54:
                    accA[1] = accA[1] + ea
                    accB[1] = accB[1] + eb
                elif k == 54:
                    accA[1] = accA[1] + jnp.where(lane < 2, ea, 0.0)
                    accA[2] = accA[2] + jnp.where(lane >= 2, ea, 0.0)
                    accB[1] = accB[1] + jnp.where(lane < 2, eb, 0.0)
                    accB[2] = accB[2] + jnp.where(lane >= 2, eb, 0.0)
                    elA = elA + jnp.where(lane == 1, ea, 0.0)
                    elB = elB + jnp.where(lane == 1, eb, 0.0)
                elif k < 62:
                    accA[2] = accA[2] + ea
                    accB[2] = accB[2] + eb
                else:
                    accA[2] = accA[2] + jnp.where(lane >= 5, ea, 0.0)
                    accB[2] = accB[2] + jnp.where(lane >= 5, eb, 0.0)
                    elA = elA + jnp.where(lane == 15, ea, 0.0)
                    elB = elB + jnp.where(lane == 15, eb, 0.0)
                cur = nxt

            sA1, sA2, sA3 = (jnp.sum(a) for a in accA)
            sB1, sB2, sB3 = (jnp.sum(a) for a in accB)
            svA = jnp.where(lane == 7, sA1, jnp.where(lane == 1, sA2, sA3))
            svB = jnp.where(lane == 7, sB1, jnp.where(lane == 1, sB2, sB3))
            rnA = 3.0 - jnp.sum(elA / svA)
            rnB = 3.0 - jnp.sum(elB / svB)
            packedA = jnp.where(lane == 0, sA1 * rnA,
                                jnp.where(lane == 1, sA2 * rnA, sA3 * rnA))
            packedB = jnp.where(lane == 0, sB1 * rnB,
                                jnp.where(lane == 1, sB2 * rnB, sB3 * rnB))
            cbuf[rA, :] = _vlog(packedA)
            cbuf[rB, :] = _vlog(packedB)

    pltpu.async_copy(in_src(0), xb0, isem0)
    pltpu.async_copy(in_src(1), xb1, isem1)

    def outer(gg, carry):
        for par in range(2):
            g = gg * 2 + par
            xbuf, obuf = xbufs[par], obufs[par]
            isem, osem = isems[par], osems[par]
            pltpu.make_async_copy(in_src(g), xbuf, isem).wait()

            @pl.when(gg >= 1)
            def _():
                pltpu.make_async_copy(obuf, out_dst(g), osem).wait()

            compute_chunk(xbuf, obuf)
            pltpu.async_copy(obuf, out_dst(g), osem)

            @pl.when(gg < (_NCH // 2 - 1))
            def _():
                pltpu.async_copy(in_src(g + 2), xbuf, isem)
        return carry

    lax.fori_loop(0, _NCH // 2, outer, 0)
    pltpu.make_async_copy(ob0, out_dst(_NCH - 2), osem0).wait()
    pltpu.make_async_copy(ob1, out_dst(_NCH - 1), osem1).wait()


def _run_sc(x, t, b):
    mesh = plsc.VectorSubcoreMesh(core_axis_name="c", subcore_axis_name="s")
    f = functools.partial(
        pl.kernel,
        mesh=mesh,
        compiler_params=pltpu.CompilerParams(needs_layout_passes=False),
        out_type=jax.ShapeDtypeStruct((_SC_ROWS, 16), jnp.float32),
        scratch_types=[
            pltpu.VMEM((_CHUNK, _N), jnp.float32),
            pltpu.VMEM((_CHUNK, _N), jnp.float32),
            pltpu.VMEM((_CHUNK, 16), jnp.float32),
            pltpu.VMEM((_CHUNK, 16), jnp.float32),
            pltpu.VMEM((_NPAD,), jnp.float32),
            pltpu.VMEM((_NPAD,), jnp.float32),
            pltpu.SemaphoreType.DMA,
            pltpu.SemaphoreType.DMA,
            pltpu.SemaphoreType.DMA,
            pltpu.SemaphoreType.DMA,
        ],
    )(_sc_kernel_body)
    return f(x, t, b)


def _tc_body(x_ref, t_ref, b_ref, o_ref):
    z = x_ref[...] * t_ref[...] + b_ref[...]
    z1 = z[:, 0:392]
    z2 = z[:, 392:866]
    z3 = z[:, 866:_N]
    m1 = jnp.max(z1, axis=1, keepdims=True)
    m2 = jnp.max(z2, axis=1, keepdims=True)
    m3 = jnp.max(z3, axis=1, keepdims=True)
    e1 = jnp.exp(z1 - m1)
    e2 = jnp.exp(z2 - m2)
    e3 = jnp.exp(z3 - m3)
    s1 = jnp.sum(e1, axis=1, keepdims=True)
    s2 = jnp.sum(e2, axis=1, keepdims=True)
    s3 = jnp.sum(e3, axis=1, keepdims=True)
    renorm = 3.0 - e1[:, -1:] / s1 - e2[:, -1:] / s2 - e3[:, -1:] / s3
    lr = jnp.log(renorm)
    c1 = m1 + jnp.log(s1) + lr
    c2 = m2 + jnp.log(s2) + lr
    c3 = m3 + jnp.log(s3) + lr
    o_ref[:, 0:391] = z1[:, :-1] - c1
    o_ref[:, 391:864] = z2[:, :-1] - c2
    o_ref[:, 864:1000] = z3[:, :-1] - c3


def _run_tc(x, t2, b2):
    nblk = _TC_ROWS // _TC_BLK
    off = _SC_ROWS // _TC_BLK
    return pl.pallas_call(
        _tc_body,
        grid=(nblk,),
        in_specs=[
            pl.BlockSpec((_TC_BLK, _N), lambda i: (i + off, 0)),
            pl.BlockSpec((1, _N), lambda i: (0, 0)),
            pl.BlockSpec((1, _N), lambda i: (0, 0)),
        ],
        out_specs=pl.BlockSpec((_TC_BLK, _NOUT), lambda i: (i + off, 0)),
        out_shape=jax.ShapeDtypeStruct((_B, _NOUT), jnp.float32),
    )(x, t2, b2)


def _tc2_body(full_ref, x_ref, t_ref, b_ref, c_ref, o_ref):
    z = x_ref[...] * t_ref[...] + b_ref[...]
    c1 = c_ref[:, 0:1]
    c2 = c_ref[:, 1:2]
    c3 = c_ref[:, 2:3]
    o_ref[:, 0:391] = z[:, 0:391] - c1
    o_ref[:, 391:864] = z[:, 392:865] - c2
    o_ref[:, 864:1000] = z[:, 866:1002] - c3


def _run_tc2(out_full, x, t2, b2, corr):
    nblk = _SC_ROWS // _TC_BLK
    return pl.pallas_call(
        _tc2_body,
        grid=(nblk,),
        in_specs=[
            pl.BlockSpec(memory_space=pltpu.MemorySpace.HBM),
            pl.BlockSpec((_TC_BLK, _N), lambda i: (i, 0)),
            pl.BlockSpec((1, _N), lambda i: (0, 0)),
            pl.BlockSpec((1, _N), lambda i: (0, 0)),
            pl.BlockSpec((_TC_BLK, 16), lambda i: (i, 0)),
        ],
        out_specs=pl.BlockSpec((_TC_BLK, _NOUT), lambda i: (i, 0)),
        out_shape=jax.ShapeDtypeStruct((_B, _NOUT), jnp.float32),
        input_output_aliases={0: 0},
    )(out_full, x, t2, b2, corr)


@jax.jit
def _run_hybrid(x, t, b, t2, b2):
    corr = _run_sc(x, t, b)
    out_full = _run_tc(x, t2, b2)
    return _run_tc2(out_full, x, t2, b2, corr)


def kernel(x, manyshotTemp, mediumshotTemp, fewshotTemp, manyshotBias,
           mediumshotBias, fewshotBias, many_mask, med_mask, few_mask):
    t2 = jnp.concatenate([manyshotTemp, mediumshotTemp, fewshotTemp], axis=1)
    b2 = jnp.concatenate([manyshotBias, mediumshotBias, fewshotBias], axis=1)
    pad = jnp.zeros((1, _NPAD - _N), jnp.float32)
    t = jnp.concatenate([t2, pad], axis=1)[0]
    b = jnp.concatenate([b2, pad], axis=1)[0]
    return _run_hybrid(x, t, b, t2, b2)
